# baseline (speedup 1.0000x reference)
import functools

import jax
import jax.numpy as jnp
import numpy as np
from jax import lax
from jax.experimental import pallas as pl
from jax.experimental.pallas import tpu as pltpu
from jax.experimental.pallas import tpu_sc as plsc

MAXLEN = 200
EMBED_DIM = 64
NUM_SEQS = 4096
NUM_WORKERS = 32
SEQ_PER_W = NUM_SEQS // NUM_WORKERS
CHUNK_SEQS = 4
CHUNK_ROWS = CHUNK_SEQS * MAXLEN
IDX_COLS = 100
IDX_ROWS_PER_CHUNK = CHUNK_ROWS // IDX_COLS
NUM_CHUNKS = SEQ_PER_W // CHUNK_SEQS
LANES = 16


def _pos_encoding_np(maxlen, embed_dim):
    position = np.arange(maxlen)[:, np.newaxis]
    div_term = np.exp(np.arange(0, embed_dim, 2) * -(np.log(10000.0) / embed_dim))
    pos_enc = np.zeros((maxlen, embed_dim), dtype=np.float32)
    pos_enc[:, 0::2] = np.sin(position * div_term)
    pos_enc[:, 1::2] = np.cos(position * div_term)
    return pos_enc


_MESH = plsc.VectorSubcoreMesh(core_axis_name="c", subcore_axis_name="s")


@functools.partial(
    pl.kernel,
    out_type=jax.ShapeDtypeStruct((NUM_SEQS, MAXLEN, EMBED_DIM), jnp.float32),
    mesh=_MESH,
    scratch_types=[
        pltpu.VMEM((IDX_ROWS_PER_CHUNK, IDX_COLS), jnp.int32),
        pltpu.VMEM((CHUNK_SEQS, MAXLEN, EMBED_DIM), jnp.float32),
        pltpu.VMEM((MAXLEN, EMBED_DIM), jnp.float32),
        pltpu.SemaphoreType.DMA,
    ],
    compiler_params=pltpu.CompilerParams(use_tc_tiling_on_sc=False),
)
def _emb_pos_kernel(idx_hbm, table_hbm, pos_hbm, out_hbm, idx_v, rows_v, pos_v, sem):
    wid = lax.axis_index("s") * 2 + lax.axis_index("c")
    pltpu.sync_copy(pos_hbm, pos_v)

    def chunk_body(c, carry):
        seq0 = wid * SEQ_PER_W + c * CHUNK_SEQS
        irow0 = seq0 * 2
        pltpu.sync_copy(idx_hbm.at[pl.ds(irow0, IDX_ROWS_PER_CHUNK)], idx_v)
        copies = []
        for j in range(IDX_ROWS_PER_CHUNK):
            dst = rows_v.at[j // 2, pl.ds((j % 2) * IDX_COLS, IDX_COLS)]
            copies.append(pltpu.async_copy(table_hbm.at[idx_v.at[j]], dst, sem))
        for cp in copies:
            cp.wait()

        def add_body(i, k):
            for sq in range(CHUNK_SEQS):
                for j in range(EMBED_DIM // LANES):
                    sl = pl.ds(j * LANES, LANES)
                    rows_v[sq, i, sl] = rows_v[sq, i, sl] + pos_v[i, sl]
            return k

        lax.fori_loop(0, MAXLEN, add_body, 0, unroll=2)
        pltpu.sync_copy(rows_v, out_hbm.at[pl.ds(seq0, CHUNK_SEQS)])
        return carry

    lax.fori_loop(0, NUM_CHUNKS, chunk_body, 0)


def kernel(x, table):
    idx2d = x.reshape(NUM_SEQS * MAXLEN // IDX_COLS, IDX_COLS)
    pos = jnp.asarray(_pos_encoding_np(MAXLEN, EMBED_DIM))
    return _emb_pos_kernel(idx2d, table, pos)


# speedup vs baseline: 1.0804x; 1.0804x over previous
import functools

import jax
import jax.numpy as jnp
import numpy as np
from jax import lax
from jax.experimental import pallas as pl
from jax.experimental.pallas import tpu as pltpu
from jax.experimental.pallas import tpu_sc as plsc

MAXLEN = 200
EMBED_DIM = 64
VOCAB = 1000000
NUM_SEQS = 4096
NUM_WORKERS = 32
BBLK = NUM_SEQS // NUM_WORKERS
LANES = 16
NREG = EMBED_DIM // LANES
TPITCH = 129


def _pos_encoding_np(maxlen, embed_dim):
    position = np.arange(maxlen)[:, np.newaxis]
    div_term = np.exp(np.arange(0, embed_dim, 2) * -(np.log(10000.0) / embed_dim))
    pos_enc = np.zeros((maxlen, embed_dim), dtype=np.float32)
    pos_enc[:, 0::2] = np.sin(position * div_term)
    pos_enc[:, 1::2] = np.cos(position * div_term)
    return pos_enc


_MESH = plsc.VectorSubcoreMesh(core_axis_name="c", subcore_axis_name="s")

JB = 512
NBLK = (VOCAB + 2 * JB - 1) // (2 * JB)
NPAIR = NBLK * JB


def _tr_body(lo_ref, hi_ref, out_ref):
    out_ref[:, 0:EMBED_DIM] = lo_ref[...].T
    out_ref[:, EMBED_DIM : 2 * EMBED_DIM] = hi_ref[...].T


def _tc_linearize(tt):
    return pl.pallas_call(
        _tr_body,
        out_shape=jax.ShapeDtypeStruct((NPAIR, 2 * EMBED_DIM), jnp.float32),
        grid=(NBLK,),
        in_specs=[
            pl.BlockSpec((EMBED_DIM, JB), lambda i: (0, 2 * i)),
            pl.BlockSpec((EMBED_DIM, JB), lambda i: (0, 2 * i + 1)),
        ],
        out_specs=pl.BlockSpec((JB, 2 * EMBED_DIM), lambda i: (i, 0)),
    )(tt, tt)


_OUT_SHAPE = (MAXLEN, EMBED_DIM // 8, NUM_SEQS // 128, 8, 128)


@functools.partial(
    pl.kernel,
    out_type=jax.ShapeDtypeStruct(_OUT_SHAPE, jnp.float32),
    mesh=_MESH,
    scratch_types=[
        pltpu.VMEM((MAXLEN, BBLK), jnp.int32),
        pltpu.VMEM((2, BBLK), jnp.int32),
        pltpu.VMEM((2, BBLK, 2 * EMBED_DIM), jnp.float32),
        pltpu.VMEM((8, 8, TPITCH), jnp.float32),
        pltpu.VMEM((MAXLEN, EMBED_DIM), jnp.float32),
        pltpu.SemaphoreType.DMA,
    ],
    compiler_params=pltpu.CompilerParams(
        use_tc_tiling_on_sc=False, needs_layout_passes=False
    ),
)
def _emb_pos_kernel(xt_hbm, tab_hbm, pos_hbm, out_hbm, idx_v, jdx_v, rows_v, t_v,
                    pos_v, gsem):
    wid = lax.axis_index("s") * 2 + lax.axis_index("c")
    b0 = wid * BBLK

    pltpu.sync_copy(pos_hbm, pos_v)
    pltpu.sync_copy(xt_hbm.at[:, pl.ds(b0, BBLK)], idx_v)

    lane = lax.iota(jnp.int32, LANES)
    td = [(lane + 16 * j) // 8 for j in range(NREG)]
    ds_ = [lane % 8 for _ in range(NREG)]

    def fill_jdx(l, buf):
        for j in range(BBLK // LANES):
            sl = pl.ds(j * LANES, LANES)
            k = idx_v[l, sl]
            jdx_v[buf, sl] = lax.shift_right_logical(k, 10) * JB + (k & (JB - 1))

    fill_jdx(0, 0)
    pltpu.async_copy(tab_hbm.at[jdx_v.at[0]], rows_v.at[0], gsem)

    def l_body(l, carry):
        cur = lax.rem(l, 2)
        nxt = lax.rem(l + 1, 2)

        @pl.when(l + 1 < MAXLEN)
        def _():
            fill_jdx(l + 1, nxt)
            pltpu.async_copy(tab_hbm.at[jdx_v.at[nxt]], rows_v.at[nxt], gsem)

        pltpu.make_async_copy(tab_hbm.at[jdx_v.at[cur]], rows_v.at[cur], gsem).wait()

        pv = tuple(pos_v[l, pl.ds(16 * j, LANES)] for j in range(NREG))

        def g_body(g, pvs):
            gb = g * LANES
            offv = (lax.shift_right_logical(idx_v[l, pl.ds(gb, LANES)], 9) & 1) * EMBED_DIM
            for i in range(LANES):
                b = gb + i
                col = jnp.full((LANES,), b, dtype=jnp.int32)
                off = offv[i]
                for j in range(NREG):
                    v = rows_v[cur, b, pl.ds(off + 16 * j, LANES)] + pvs[j]
                    plsc.store_scatter(t_v, [td[j], ds_[j], col], v)
            return pvs

        lax.fori_loop(0, BBLK // LANES, g_body, pv)
        pltpu.sync_copy(t_v.at[:, :, pl.ds(0, 128)], out_hbm.at[l, :, wid])
        return carry

    lax.fori_loop(0, MAXLEN, l_body, 0)


def kernel(x, table):
    xt = x.T
    tab2 = _tc_linearize(table.T)
    pos = jnp.asarray(_pos_encoding_np(MAXLEN, EMBED_DIM))
    k5 = _emb_pos_kernel(xt, tab2, pos)
    return k5.transpose(2, 4, 0, 1, 3).reshape(NUM_SEQS, MAXLEN, EMBED_DIM)
